# baseline (speedup 1.0000x reference)
import functools

import jax
import jax.numpy as jnp
from jax import lax
from jax.experimental import pallas as pl
from jax.experimental.pallas import tpu as pltpu
from jax.experimental.pallas import tpu_sc as plsc

_LANES = 16
_CW = 128


def _interaction_table_kernel(cw_ref, sw_ref, cb_ref, sb_ref, g_ref):
    v, d = cw_ref.shape
    chunks = g_ref.shape[0] // v
    dn = (((1,), (1,)), ((), ()))
    t = lax.dot_general(cw_ref[...], sw_ref[...], dn,
                        preferred_element_type=jnp.float32,
                        precision=lax.Precision.HIGHEST)
    t = t + cb_ref[...] + sb_ref[...]
    for k in range(chunks):
        g_ref[k * v:(k + 1) * v, :] = t[:, k * _CW:(k + 1) * _CW]


def _build_interaction_table(c_weight, s_weight, c_biase, s_biase):
    v, d = c_weight.shape
    chunks = (v + _CW - 1) // _CW
    steps = 4
    cps = chunks // steps
    return pl.pallas_call(
        _interaction_table_kernel,
        grid=(steps,),
        in_specs=[
            pl.BlockSpec((v, d), lambda j: (0, 0)),
            pl.BlockSpec((cps * _CW, d), lambda j: (j, 0)),
            pl.BlockSpec((v, 1), lambda j: (0, 0)),
            pl.BlockSpec((1, cps * _CW), lambda j: (0, j)),
        ],
        out_specs=pl.BlockSpec((cps * v, _CW), lambda j: (j, 0)),
        out_shape=jax.ShapeDtypeStruct((chunks * v, _CW), jnp.float32),
    )(c_weight, s_weight, c_biase, s_biase.reshape(1, v))


def _make_sc_gather(v, b, num_workers, chunk):
    per_w = b // num_workers
    rows = per_w // chunk
    mesh = plsc.VectorSubcoreMesh(core_axis_name="c", subcore_axis_name="s")

    @functools.partial(
        pl.kernel,
        mesh=mesh,
        out_type=jax.ShapeDtypeStruct((b,), jnp.float32),
        scratch_types=[
            pltpu.VMEM((per_w,), jnp.int32),
            pltpu.VMEM((per_w,), jnp.int32),
            pltpu.VMEM((per_w,), jnp.int32),
            pltpu.VMEM((per_w,), jnp.float32),
            pltpu.SemaphoreType.DMA,
            pltpu.SemaphoreType.DMA,
        ],
    )
    def sc_gather(g_hbm, c_hbm, s_hbm, out_hbm, c_v, s_v, idx_v, val_v,
                  semi, sem):
        wid = lax.axis_index("s") * 2 + lax.axis_index("c")
        base = wid * per_w
        stage = (
            pltpu.async_copy(c_hbm.at[pl.ds(base, per_w)], c_v, semi),
            pltpu.async_copy(s_hbm.at[pl.ds(base, per_w)], s_v, semi),
        )
        for cp in stage:
            cp.wait()

        def idx_body(i, carry):
            sl = pl.ds(i * _LANES, _LANES)
            sv = s_v[sl]
            idx_v[sl] = (
                lax.shift_right_logical(sv, 7) * (_CW * v)
                + lax.shift_left(c_v[sl], 7)
                + (sv & (_CW - 1))
            )
            return carry

        copies = []
        for r in range(rows):
            lax.fori_loop(r * (chunk // _LANES), (r + 1) * (chunk // _LANES),
                          idx_body, 0)
            copies.append(pltpu.async_copy(
                g_hbm.at[idx_v.at[pl.ds(r * chunk, chunk)]],
                val_v.at[pl.ds(r * chunk, chunk)],
                sem,
            ))
        for cp in copies:
            cp.wait()
        pltpu.sync_copy(val_v, out_hbm.at[pl.ds(base, per_w)])

    return sc_gather


def kernel(c, s, c_weight, c_biase, s_weight, s_biase):
    v, _ = c_weight.shape
    b = c.shape[0]

    g = _build_interaction_table(c_weight, s_weight, c_biase, s_biase)
    g_flat = g.reshape(g.shape[0] * _CW)

    out = _make_sc_gather(v, b, 32, 128)(
        g_flat, c.astype(jnp.int32), s.astype(jnp.int32))
    return out.reshape(b, 1)


# speedup vs baseline: 1.0107x; 1.0107x over previous
import functools

import jax
import jax.numpy as jnp
from jax import lax
from jax.experimental import pallas as pl
from jax.experimental.pallas import tpu as pltpu
from jax.experimental.pallas import tpu_sc as plsc

_LANES = 16
_CW = 128


def _interaction_table_kernel(cw_ref, sw_ref, cb_ref, sb_ref, g_ref):
    v, d = cw_ref.shape
    chunks = g_ref.shape[0] // v
    dn = (((1,), (1,)), ((), ()))
    t = lax.dot_general(cw_ref[...], sw_ref[...], dn,
                        preferred_element_type=jnp.float32,
                        precision=lax.Precision.HIGHEST)
    t = t + cb_ref[...] + sb_ref[...]
    for k in range(chunks):
        g_ref[k * v:(k + 1) * v, :] = t[:, k * _CW:(k + 1) * _CW]


def _build_interaction_table(c_weight, s_weight, c_biase, s_biase):
    v, d = c_weight.shape
    chunks = (v + _CW - 1) // _CW
    steps = 2
    cps = chunks // steps
    return pl.pallas_call(
        _interaction_table_kernel,
        grid=(steps,),
        in_specs=[
            pl.BlockSpec((v, d), lambda j: (0, 0)),
            pl.BlockSpec((cps * _CW, d), lambda j: (j, 0)),
            pl.BlockSpec((v, 1), lambda j: (0, 0)),
            pl.BlockSpec((1, cps * _CW), lambda j: (0, j)),
        ],
        out_specs=pl.BlockSpec((cps * v, _CW), lambda j: (j, 0)),
        out_shape=jax.ShapeDtypeStruct((chunks * v, _CW), jnp.float32),
    )(c_weight, s_weight, c_biase, s_biase.reshape(1, v))


def _make_sc_gather(v, b, num_workers, chunk):
    per_w = b // num_workers
    rows = per_w // chunk
    mesh = plsc.VectorSubcoreMesh(core_axis_name="c", subcore_axis_name="s")

    @functools.partial(
        pl.kernel,
        mesh=mesh,
        out_type=jax.ShapeDtypeStruct((b,), jnp.float32),
        scratch_types=[
            pltpu.VMEM((per_w,), jnp.int32),
            pltpu.VMEM((per_w,), jnp.int32),
            pltpu.VMEM((per_w,), jnp.int32),
            pltpu.VMEM((per_w,), jnp.float32),
            pltpu.SemaphoreType.DMA,
            pltpu.SemaphoreType.DMA,
        ],
    )
    def sc_gather(g_hbm, c_hbm, s_hbm, out_hbm, c_v, s_v, idx_v, val_v,
                  semi, sem):
        wid = lax.axis_index("s") * 2 + lax.axis_index("c")
        base = wid * per_w
        stage = (
            pltpu.async_copy(c_hbm.at[pl.ds(base, per_w)], c_v, semi),
            pltpu.async_copy(s_hbm.at[pl.ds(base, per_w)], s_v, semi),
        )
        for cp in stage:
            cp.wait()

        def idx_body(i, carry):
            sl = pl.ds(i * _LANES, _LANES)
            sv = s_v[sl]
            idx_v[sl] = (
                lax.shift_right_logical(sv, 7) * (_CW * v)
                + lax.shift_left(c_v[sl], 7)
                + (sv & (_CW - 1))
            )
            return carry

        copies = []
        for r in range(rows):
            lax.fori_loop(r * (chunk // _LANES), (r + 1) * (chunk // _LANES),
                          idx_body, 0)
            copies.append(pltpu.async_copy(
                g_hbm.at[idx_v.at[pl.ds(r * chunk, chunk)]],
                val_v.at[pl.ds(r * chunk, chunk)],
                sem,
            ))
        for cp in copies:
            cp.wait()
        pltpu.sync_copy(val_v, out_hbm.at[pl.ds(base, per_w)])

    return sc_gather


def kernel(c, s, c_weight, c_biase, s_weight, s_biase):
    v, _ = c_weight.shape
    b = c.shape[0]

    g = _build_interaction_table(c_weight, s_weight, c_biase, s_biase)
    g_flat = g.reshape(g.shape[0] * _CW)

    out = _make_sc_gather(v, b, 32, 128)(
        g_flat, c.astype(jnp.int32), s.astype(jnp.int32))
    return out.reshape(b, 1)


# speedup vs baseline: 1.0120x; 1.0013x over previous
import functools

import jax
import jax.numpy as jnp
from jax import lax
from jax.experimental import pallas as pl
from jax.experimental.pallas import tpu as pltpu
from jax.experimental.pallas import tpu_sc as plsc

_LANES = 16
_CW = 128


def _interaction_table_kernel(cw_ref, sw_ref, cb_ref, sb_ref, g_ref):
    v, d = cw_ref.shape
    chunks = g_ref.shape[0] // v
    dn = (((1,), (1,)), ((), ()))
    t = lax.dot_general(cw_ref[...], sw_ref[...], dn,
                        preferred_element_type=jnp.float32,
                        precision=lax.Precision.HIGHEST)
    t = t + cb_ref[...] + sb_ref[...]
    for k in range(chunks):
        g_ref[k * v:(k + 1) * v, :] = t[:, k * _CW:(k + 1) * _CW]


def _build_interaction_table(c_weight, s_weight, c_biase, s_biase):
    v, d = c_weight.shape
    chunks = (v + _CW - 1) // _CW
    steps = 2
    cps = chunks // steps
    return pl.pallas_call(
        _interaction_table_kernel,
        grid=(steps,),
        in_specs=[
            pl.BlockSpec((v, d), lambda j: (0, 0)),
            pl.BlockSpec((cps * _CW, d), lambda j: (j, 0)),
            pl.BlockSpec((v, 1), lambda j: (0, 0)),
            pl.BlockSpec((1, cps * _CW), lambda j: (0, j)),
        ],
        out_specs=pl.BlockSpec((cps * v, _CW), lambda j: (j, 0)),
        out_shape=jax.ShapeDtypeStruct((chunks * v, _CW), jnp.float32),
    )(c_weight, s_weight, c_biase, s_biase.reshape(1, v))


def _make_sc_gather(v, b, num_workers, chunk):
    per_w = b // num_workers
    rows = per_w // chunk
    mesh = plsc.VectorSubcoreMesh(core_axis_name="c", subcore_axis_name="s")

    @functools.partial(
        pl.kernel,
        mesh=mesh,
        out_type=jax.ShapeDtypeStruct((b,), jnp.float32),
        scratch_types=[
            pltpu.VMEM((per_w,), jnp.int32),
            pltpu.VMEM((per_w,), jnp.int32),
            pltpu.VMEM((per_w,), jnp.int32),
            pltpu.VMEM((per_w,), jnp.float32),
            pltpu.SemaphoreType.DMA,
            pltpu.SemaphoreType.DMA,
        ],
    )
    def sc_gather(g_hbm, c_hbm, s_hbm, out_hbm, c_v, s_v, idx_v, val_v,
                  semi, sem):
        wid = lax.axis_index("s") * 2 + lax.axis_index("c")
        base = wid * per_w
        stage = (
            pltpu.async_copy(c_hbm.at[pl.ds(base, per_w)], c_v, semi),
            pltpu.async_copy(s_hbm.at[pl.ds(base, per_w)], s_v, semi),
        )
        for cp in stage:
            cp.wait()

        def idx_body(i, carry):
            sl = pl.ds(i * _LANES, _LANES)
            sv = s_v[sl]
            idx_v[sl] = (
                lax.shift_right_logical(sv, 7) * (_CW * v)
                + lax.shift_left(c_v[sl], 7)
                + (sv & (_CW - 1))
            )
            return carry

        copies = []
        for r in range(rows):
            lax.fori_loop(r * (chunk // _LANES), (r + 1) * (chunk // _LANES),
                          idx_body, 0)
            copies.append(pltpu.async_copy(
                g_hbm.at[idx_v.at[pl.ds(r * chunk, chunk)]],
                val_v.at[pl.ds(r * chunk, chunk)],
                sem,
            ))
        for cp in copies:
            cp.wait()
        pltpu.sync_copy(val_v, out_hbm.at[pl.ds(base, per_w)])

    return sc_gather


def kernel(c, s, c_weight, c_biase, s_weight, s_biase):
    v, _ = c_weight.shape
    b = c.shape[0]

    g = _build_interaction_table(c_weight, s_weight, c_biase, s_biase)
    g_flat = g.reshape(g.shape[0] * _CW)

    out = _make_sc_gather(v, b, 32, 256)(
        g_flat, c.astype(jnp.int32), s.astype(jnp.int32))
    return out.reshape(b, 1)
